# baseline (speedup 1.0000x reference)
import functools

import jax
import jax.numpy as jnp
from jax import lax
from jax.experimental import pallas as pl
from jax.experimental.pallas import tpu as pltpu
from jax.experimental.pallas import tpu_sc as plsc

_NC = 2
_NS = 16
_NW = _NC * _NS
_L = 16

_CB = 128
_NSTREAM = _CB // 128
_GROUPS = _CB // _L
_LEAD = 2
_NBUF = 4


def _rsqrt(v):
    b = plsc.bitcast(v, jnp.int32)
    y = plsc.bitcast(jnp.int32(0x5F3759DF) - (b >> 1), jnp.float32)
    for _ in range(3):
        y = y * (1.5 - 0.5 * v * y * y)
    return y


@functools.lru_cache(maxsize=None)
def _build_embed_ln(ntok, vocab, embed):
    assert ntok % (_NW * _CB) == 0 and embed == 64
    tokw = ntok // _NW
    nch = tokw // _CB
    slabs_w = tokw // 128
    assert nch % _NBUF == 0
    mesh = plsc.VectorSubcoreMesh(core_axis_name="c", subcore_axis_name="s")

    def body(
        x2, table, w_h, b_h, out, idxs, rowss, slabss, w_v, b_v, gsems, ssems,
        isems,
    ):
        wid = lax.axis_index("s") * _NC + lax.axis_index("c")
        xrow0 = wid * slabs_w

        pltpu.sync_copy(w_h, w_v)
        pltpu.sync_copy(b_h, b_v)

        def idx_desc(g, b):
            return pltpu.make_async_copy(
                x2.at[pl.ds(xrow0 + g * _NSTREAM, _NSTREAM)], idxs[b], isems[b]
            )

        def gather_descs(b):
            return [
                pltpu.make_async_copy(
                    table.at[idxs[b].at[j]],
                    rowss[b].at[pl.ds(j * 128, 128), :],
                    gsems[b],
                )
                for j in range(_NSTREAM)
            ]

        def fire(b):
            for d in gather_descs(b):
                d.start()

        def wait_gather(b):
            for d in gather_descs(b):
                d.wait()

        def store_descs(g, b):
            return [
                pltpu.make_async_copy(
                    slabss[b],
                    out.at[pl.ds(xrow0 + g * _NSTREAM, _NSTREAM)],
                    ssems[b],
                )
            ]

        def start_store(g, b):
            for d in store_descs(g, b):
                d.start()

        def wait_store(b):
            for d in store_descs(0, b):
                d.wait()

        def compute(b):
            rows = rowss[b]
            slabs = slabss[b]
            nk = embed // _L
            wv = [w_v[pl.ds(k * _L, _L)] for k in range(nk)]
            bv = [b_v[pl.ds(k * _L, _L)] for k in range(nk)]
            iota16 = lax.iota(jnp.int32, _L)
            onehot = [iota16 == r for r in range(_L)]

            def group(i, carry):
                r0 = i * _L
                acc_s = jnp.zeros((_L,), jnp.float32)
                acc_q = jnp.zeros((_L,), jnp.float32)
                for r in range(_L):
                    v = [rows[r0 + r, pl.ds(k * _L, _L)] for k in range(nk)]
                    s = (v[0] + v[1]) + (v[2] + v[3])
                    q = (v[0] * v[0] + v[1] * v[1]) + (
                        v[2] * v[2] + v[3] * v[3]
                    )
                    acc_s = jnp.where(onehot[r], jnp.sum(s), acc_s)
                    acc_q = jnp.where(onehot[r], jnp.sum(q), acc_q)
                mean = acc_s * (1.0 / embed)
                var = acc_q * (1.0 / embed) - mean * mean
                inv = _rsqrt(var + 1e-5)
                half = i * (_L // 2)
                for r in range(_L):
                    m = mean[r]
                    a = inv[r]
                    brow = half + r // 2
                    for k in range(nk):
                        v = rows[r0 + r, pl.ds(k * _L, _L)]
                        slabs[
                            brow >> 6,
                            brow & 63,
                            pl.ds((r % 2) * 64 + k * _L, _L),
                        ] = (v - m) * a * wv[k] + bv[k]
                return carry

            lax.fori_loop(0, _GROUPS, group, 0)

        for c in range(_LEAD):
            idx_desc(c, c).start()
        for c in range(_LEAD):
            idx_desc(c, c).wait()
            fire(c)
        idx_desc(_LEAD, _LEAD).start()

        def step(h, carry):
            for par in range(_NBUF):
                g = h * _NBUF + par
                nxt = g + _LEAD
                pre = g + _LEAD + 1
                b_nxt = (par + _LEAD) % _NBUF
                b_pre = (par + _LEAD + 1) % _NBUF

                @pl.when(pre < nch)
                def _():
                    idx_desc(pre, b_pre).start()

                @pl.when(nxt < nch)
                def _():
                    idx_desc(nxt, b_nxt).wait()
                    fire(b_nxt)

                wait_gather(par)

                @pl.when(g >= _NBUF)
                def _():
                    wait_store(par)

                compute(par)
                start_store(g, par)
            return carry

        lax.fori_loop(0, nch // _NBUF, step, 0)
        for b in range(_NBUF):
            wait_store(b)

    idx_t = pltpu.VMEM((_NSTREAM, 128), jnp.int32)
    rows_t = pltpu.VMEM((_CB, 64), jnp.float32)
    slab_t = pltpu.VMEM((_NSTREAM, 64, 128), jnp.float32)
    return pl.kernel(
        body,
        out_type=jax.ShapeDtypeStruct((ntok // 128, 64, 128), jnp.float32),
        mesh=mesh,
        compiler_params=pltpu.CompilerParams(
            needs_layout_passes=False, use_tc_tiling_on_sc=False
        ),
        scratch_types=[
            [idx_t] * _NBUF,
            [rows_t] * _NBUF,
            [slab_t] * _NBUF,
            pltpu.VMEM((embed,), jnp.float32),
            pltpu.VMEM((embed,), jnp.float32),
            [pltpu.SemaphoreType.DMA] * _NBUF,
            [pltpu.SemaphoreType.DMA] * _NBUF,
            [pltpu.SemaphoreType.DMA] * _NBUF,
        ],
    )


def _mask_body(x_ref, o_ref):
    o_ref[...] = x_ref[...] > 0


def kernel(x, table, ln_weight, ln_bias):
    bsz, seq = x.shape
    ntok = bsz * seq
    embed = table.shape[1]
    x2 = x.reshape(ntok // 128, 128)
    out = _build_embed_ln(ntok, table.shape[0], embed)(
        x2, table, ln_weight, ln_bias
    ).reshape(bsz, seq, embed)
    mask = pl.pallas_call(
        _mask_body,
        out_shape=jax.ShapeDtypeStruct((bsz, seq), jnp.bool_),
    )(x)
    return out, mask


# speedup vs baseline: 1.5758x; 1.5758x over previous
import functools

import jax
import jax.numpy as jnp
from jax import lax
from jax.experimental import pallas as pl
from jax.experimental.pallas import tpu as pltpu
from jax.experimental.pallas import tpu_sc as plsc

_NC = 2
_NS = 16
_NW = _NC * _NS
_L = 16

_CB = 128
_GROUPS = _CB // _L
_LEAD = 3
_NBUF = 4
_IBLK = 8


def _rsqrt(v):
    b = plsc.bitcast(v, jnp.int32)
    y = plsc.bitcast(jnp.int32(0x5F3759DF) - (b >> 1), jnp.float32)
    for _ in range(3):
        y = y * (1.5 - 0.5 * v * y * y)
    return y


@functools.lru_cache(maxsize=None)
def _build_embed_ln(bsz, seq, vocab, embed):
    assert bsz == _NW * _CB and embed == 64 and seq % _NBUF == 0
    nch = seq
    mesh = plsc.VectorSubcoreMesh(core_axis_name="c", subcore_axis_name="s")

    def body(xt, table, w_h, b_h, out, midx, rowss, slabss, w_v, b_v, gsems,
             ssems, isems):
        wid = lax.axis_index("s") * _NC + lax.axis_index("c")
        b0 = wid * _CB

        pltpu.sync_copy(w_h, w_v)
        pltpu.sync_copy(b_h, b_v)

        def iblk_desc(blk, slot):
            return pltpu.make_async_copy(
                xt.at[pl.ds(blk * _IBLK, _IBLK), pl.ds(b0, _CB)],
                midx.at[slot],
                isems[slot],
            )

        def iblk_both(blk, fn):
            @pl.when(blk % 2 == 0)
            def _():
                fn(iblk_desc(blk, 0))

            @pl.when(blk % 2 == 1)
            def _():
                fn(iblk_desc(blk, 1))

        def gather_descs(g, b):
            return [
                pltpu.make_async_copy(
                    table.at[midx.at[(g // _IBLK) & 1].at[g % _IBLK]],
                    rowss[b],
                    gsems[b],
                )
            ]

        def fire(g, b):
            for d in gather_descs(g, b):
                d.start()

        def wait_gather(g, b):
            for d in gather_descs(g, b):
                d.wait()

        def store_descs(g, b):
            return [
                pltpu.make_async_copy(
                    slabss[b], out.at[g, :, wid], ssems[b]
                )
            ]

        def start_store(g, b):
            for d in store_descs(g, b):
                d.start()

        def wait_store(b):
            for d in store_descs(0, b):
                d.wait()

        def compute(b):
            rows = rowss[b]
            slabs = slabss[b]
            nk = embed // _L
            wv = [w_v[pl.ds(k * _L, _L)] for k in range(nk)]
            bv = [b_v[pl.ds(k * _L, _L)] for k in range(nk)]
            iota16 = lax.iota(jnp.int32, _L)
            onehot = [iota16 == r for r in range(_L)]
            xors = [iota16 ^ d for d in range(_L)]

            def group(i, carry):
                r0 = i * _L
                acc_s = jnp.zeros((_L,), jnp.float32)
                acc_q = jnp.zeros((_L,), jnp.float32)
                for r in range(_L):
                    v = [rows[r0 + r, pl.ds(k * _L, _L)] for k in range(nk)]
                    s = (v[0] + v[1]) + (v[2] + v[3])
                    q = (v[0] * v[0] + v[1] * v[1]) + (
                        v[2] * v[2] + v[3] * v[3]
                    )
                    acc_s = jnp.where(onehot[r], jnp.sum(s), acc_s)
                    acc_q = jnp.where(onehot[r], jnp.sum(q), acc_q)
                mean = acc_s * (1.0 / embed)
                var = acc_q * (1.0 / embed) - mean * mean
                inv = _rsqrt(var + 1e-5)
                for r in range(_L):
                    m = mean[r]
                    a = inv[r]
                    for k in range(nk):
                        v = rows[r0 + r, pl.ds(k * _L, _L)]
                        rows[r0 + r, pl.ds(k * _L, _L)] = (
                            (v - m) * a * wv[k] + bv[k]
                        )
                return carry

            lax.fori_loop(0, _GROUPS, group, 0)

            def tblock(bi, carry):
                ridx = iota16 + bi * _L
                for ej in range(embed // _L):
                    for d in range(_L):
                        cidx = xors[d] + ej * _L
                        v = plsc.load_gather(rows, [ridx, cidx])
                        plsc.store_scatter(
                            slabs, [cidx >> 3, cidx & 7, ridx], v
                        )
                return carry

            lax.fori_loop(0, _GROUPS, tblock, 0)

        iblk_desc(0, 0).start()
        iblk_desc(0, 0).wait()
        iblk_desc(1, 1).start()
        for c in range(_LEAD):
            fire(c, c)

        nblk = nch // _IBLK

        def step(h, carry):
            for par in range(_NBUF):
                g = h * _NBUF + par
                nxt = g + _LEAD
                b_nxt = (par + _LEAD) % _NBUF

                @pl.when(nxt < nch)
                def _():
                    @pl.when(nxt % _IBLK == 0)
                    def _():
                        iblk_both(nxt // _IBLK, lambda d: d.wait())

                    fire(nxt, b_nxt)

                wait_gather(g, par)

                @pl.when(g % _IBLK == _IBLK - 1)
                def _():
                    blk = g // _IBLK + 2

                    @pl.when(blk < nblk)
                    def _():
                        iblk_both(blk, lambda d: d.start())

                @pl.when(g >= _NBUF)
                def _():
                    wait_store(par)

                compute(par)
                start_store(g, par)
            return carry

        lax.fori_loop(0, nch // _NBUF, step, 0)
        for b in range(_NBUF):
            wait_store(b)

    midx_t = pltpu.VMEM((2, _IBLK, _CB), jnp.int32)
    rows_t = pltpu.VMEM((_CB, 64), jnp.float32)
    slab_t = pltpu.VMEM((8, 8, 128), jnp.float32)
    return pl.kernel(
        body,
        out_type=jax.ShapeDtypeStruct(
            (seq, embed // 8, bsz // 128, 8, 128), jnp.float32
        ),
        mesh=mesh,
        compiler_params=pltpu.CompilerParams(
            needs_layout_passes=False, use_tc_tiling_on_sc=False
        ),
        scratch_types=[
            midx_t,
            [rows_t] * _NBUF,
            [slab_t] * _NBUF,
            pltpu.VMEM((embed,), jnp.float32),
            pltpu.VMEM((embed,), jnp.float32),
            [pltpu.SemaphoreType.DMA] * _NBUF,
            [pltpu.SemaphoreType.DMA] * _NBUF,
            [pltpu.SemaphoreType.DMA] * 2,
        ],
    )


def _mask_body(x_ref, o_ref):
    o_ref[...] = x_ref[...] > 0


def kernel(x, table, ln_weight, ln_bias):
    bsz, seq = x.shape
    embed = table.shape[1]
    xt = jnp.transpose(x)
    out5 = _build_embed_ln(bsz, seq, table.shape[0], embed)(
        xt, table, ln_weight, ln_bias
    )
    out = jnp.transpose(out5, (2, 4, 0, 1, 3)).reshape(bsz, seq, embed)
    mask = pl.pallas_call(
        _mask_body,
        out_shape=jax.ShapeDtypeStruct((bsz, seq), jnp.bool_),
    )(x)
    return out, mask


# speedup vs baseline: 1.9360x; 1.2286x over previous
import functools

import jax
import jax.numpy as jnp
from jax import lax
from jax.experimental import pallas as pl
from jax.experimental.pallas import tpu as pltpu
from jax.experimental.pallas import tpu_sc as plsc

_NC = 2
_NS = 16
_NW = _NC * _NS
_L = 16

_CB = 128
_GROUPS = _CB // _L
_LEAD = 2
_NBUF = 4


def _rsqrt(v):
    b = plsc.bitcast(v, jnp.int32)
    y = plsc.bitcast(jnp.int32(0x5F3759DF) - (b >> 1), jnp.float32)
    for _ in range(3):
        y = y * (1.5 - 0.5 * v * y * y)
    return y


@functools.lru_cache(maxsize=None)
def _build_embed_ln(bsz, seq, vocab, embed):
    assert bsz == _NW * _CB and embed == 64 and seq % _NBUF == 0
    nch = seq
    mesh = plsc.VectorSubcoreMesh(core_axis_name="c", subcore_axis_name="s")

    def body(xt, table, w_h, b_h, out, idxs, rowss, slabss, w_v, b_v, gsems,
             ssems, isems):
        wid = lax.axis_index("s") * _NC + lax.axis_index("c")
        b0 = wid * _CB

        pltpu.sync_copy(w_h, w_v)
        pltpu.sync_copy(b_h, b_v)

        def idx_desc(g, b):
            return pltpu.make_async_copy(
                xt.at[pl.ds(g, 1), pl.ds(b0, _CB)], idxs[b], isems[b]
            )

        def gather_descs(b):
            return [
                pltpu.make_async_copy(
                    table.at[idxs[b].at[0]], rowss[b], gsems[b]
                )
            ]

        def fire(b):
            for d in gather_descs(b):
                d.start()

        def wait_gather(b):
            for d in gather_descs(b):
                d.wait()

        def store_descs(g, b):
            return [
                pltpu.make_async_copy(
                    slabss[b], out.at[g, :, wid], ssems[b]
                )
            ]

        def start_store(g, b):
            for d in store_descs(g, b):
                d.start()

        def wait_store(b):
            for d in store_descs(0, b):
                d.wait()

        def compute(b):
            rows = rowss[b]
            slabs = slabss[b]
            nk = embed // _L
            wv = [w_v[pl.ds(k * _L, _L)] for k in range(nk)]
            bv = [b_v[pl.ds(k * _L, _L)] for k in range(nk)]
            iota16 = lax.iota(jnp.int32, _L)
            onehot = [iota16 == r for r in range(_L)]
            xors = [iota16 ^ d for d in range(_L)]

            def group(i, carry):
                r0 = i * _L
                acc_s = jnp.zeros((_L,), jnp.float32)
                acc_q = jnp.zeros((_L,), jnp.float32)
                for r in range(_L):
                    v = [rows[r0 + r, pl.ds(k * _L, _L)] for k in range(nk)]
                    s = (v[0] + v[1]) + (v[2] + v[3])
                    q = (v[0] * v[0] + v[1] * v[1]) + (
                        v[2] * v[2] + v[3] * v[3]
                    )
                    acc_s = jnp.where(onehot[r], jnp.sum(s), acc_s)
                    acc_q = jnp.where(onehot[r], jnp.sum(q), acc_q)
                mean = acc_s * (1.0 / embed)
                var = acc_q * (1.0 / embed) - mean * mean
                inv = _rsqrt(var + 1e-5)
                for r in range(_L):
                    m = mean[r]
                    a = inv[r]
                    for k in range(nk):
                        v = rows[r0 + r, pl.ds(k * _L, _L)]
                        rows[r0 + r, pl.ds(k * _L, _L)] = (
                            (v - m) * a * wv[k] + bv[k]
                        )
                return carry

            lax.fori_loop(0, _GROUPS, group, 0)

            def tblock(blk, carry):
                bi = blk // (embed // _L)
                ej = blk % (embed // _L)
                ridx = iota16 + bi * _L
                for d in range(_L):
                    cidx = xors[d] + ej * _L
                    v = plsc.load_gather(rows, [ridx, cidx])
                    plsc.store_scatter(
                        slabs, [cidx >> 3, cidx & 7, ridx], v
                    )
                return carry

            lax.fori_loop(0, _GROUPS * (embed // _L), tblock, 0)

        for c in range(_LEAD):
            idx_desc(c, c).start()
        for c in range(_LEAD):
            idx_desc(c, c).wait()
            fire(c)
        idx_desc(_LEAD, _LEAD).start()

        def step(h, carry):
            for par in range(_NBUF):
                g = h * _NBUF + par
                nxt = g + _LEAD
                pre = g + _LEAD + 1
                b_nxt = (par + _LEAD) % _NBUF
                b_pre = (par + _LEAD + 1) % _NBUF

                @pl.when(pre < nch)
                def _():
                    idx_desc(pre, b_pre).start()

                @pl.when(nxt < nch)
                def _():
                    idx_desc(nxt, b_nxt).wait()
                    fire(b_nxt)

                wait_gather(par)

                @pl.when(g >= _NBUF)
                def _():
                    wait_store(par)

                compute(par)
                start_store(g, par)
            return carry

        lax.fori_loop(0, nch // _NBUF, step, 0)
        for b in range(_NBUF):
            wait_store(b)

    idx_t = pltpu.VMEM((1, _CB), jnp.int32)
    rows_t = pltpu.VMEM((_CB, 64), jnp.float32)
    slab_t = pltpu.VMEM((8, 8, 128), jnp.float32)
    return pl.kernel(
        body,
        out_type=jax.ShapeDtypeStruct(
            (seq, embed // 8, bsz // 128, 8, 128), jnp.float32
        ),
        mesh=mesh,
        compiler_params=pltpu.CompilerParams(
            needs_layout_passes=False, use_tc_tiling_on_sc=False
        ),
        scratch_types=[
            [idx_t] * _NBUF,
            [rows_t] * _NBUF,
            [slab_t] * _NBUF,
            pltpu.VMEM((embed,), jnp.float32),
            pltpu.VMEM((embed,), jnp.float32),
            [pltpu.SemaphoreType.DMA] * _NBUF,
            [pltpu.SemaphoreType.DMA] * _NBUF,
            [pltpu.SemaphoreType.DMA] * _NBUF,
        ],
    )


def _mask_body(x_ref, o_ref):
    o_ref[...] = x_ref[...] > 0


def kernel(x, table, ln_weight, ln_bias):
    bsz, seq = x.shape
    embed = table.shape[1]
    xt = jnp.transpose(x)
    out5 = _build_embed_ln(bsz, seq, table.shape[0], embed)(
        xt, table, ln_weight, ln_bias
    )
    out = jnp.transpose(out5, (2, 4, 0, 1, 3)).reshape(bsz, seq, embed)
    mask = pl.pallas_call(
        _mask_body,
        out_shape=jax.ShapeDtypeStruct((bsz, seq), jnp.bool_),
    )(x)
    return out, mask
